# baseline (speedup 1.0000x reference)
import functools

import jax
import jax.numpy as jnp
from jax.experimental import pallas as pl
from jax.experimental.pallas import tpu as pltpu

_TOKENS = 16384
_D = 2048
_E = 64
_T = 512


def _router_body(x_ref, w_ref, b_ref, tw_ref, ti_ref):
    logits = jax.lax.dot_general(
        x_ref[...], w_ref[...], (((1,), (1,)), ((), ())),
        preferred_element_type=jnp.float32)
    logits = logits + b_ref[...]
    m = jnp.max(logits, axis=-1, keepdims=True)
    e = jnp.exp(logits - m)
    p = e / jnp.sum(e, axis=-1, keepdims=True)

    col = jax.lax.broadcasted_iota(jnp.int32, p.shape, 1)
    m1 = jnp.max(p, axis=-1, keepdims=True)
    i1 = jnp.min(jnp.where(p == m1, col, _E), axis=-1, keepdims=True)
    masked = jnp.where(col == i1, -1.0, p)
    m2 = jnp.max(masked, axis=-1, keepdims=True)
    i2 = jnp.min(jnp.where(masked == m2, col, _E), axis=-1, keepdims=True)

    tw_ref[...] = jnp.concatenate([m1, m2], axis=-1)
    ti_ref[...] = jnp.concatenate([i1, i2], axis=-1)


@functools.partial(jax.jit, static_argnames=())
def _router(x, W, b):
    grid = (_TOKENS // _T,)
    return pl.pallas_call(
        _router_body,
        grid=grid,
        in_specs=[
            pl.BlockSpec((_T, _D), lambda i: (i, 0)),
            pl.BlockSpec((_E, _D), lambda i: (0, 0)),
            pl.BlockSpec((1, _E), lambda i: (0, 0)),
        ],
        out_specs=[
            pl.BlockSpec((_T, 2), lambda i: (i, 0)),
            pl.BlockSpec((_T, 2), lambda i: (i, 0)),
        ],
        out_shape=[
            jax.ShapeDtypeStruct((_TOKENS, 2), jnp.float32),
            jax.ShapeDtypeStruct((_TOKENS, 2), jnp.int32),
        ],
    )(x, W, b.reshape(1, _E))


def kernel(x, k, W, b):
    tw, ti = _router(x, W, b)
    ti = ti + (jnp.asarray(k, dtype=ti.dtype) - 2)
    return (tw, ti)


# speedup vs baseline: 1.2183x; 1.2183x over previous
import functools

import jax
import jax.numpy as jnp
from jax.experimental import pallas as pl
from jax.experimental.pallas import tpu as pltpu

_TOKENS = 16384
_D = 2048
_E = 64
_T = 2048


def _router_body(x_ref, w_ref, b_ref, tw_ref, ti_ref):
    logits = jax.lax.dot_general(
        x_ref[...], w_ref[...], (((1,), (1,)), ((), ())),
        preferred_element_type=jnp.float32)
    logits = logits + b_ref[...]
    m = jnp.max(logits, axis=-1, keepdims=True)
    e = jnp.exp(logits - m)
    p = e / jnp.sum(e, axis=-1, keepdims=True)

    col = jax.lax.broadcasted_iota(jnp.int32, p.shape, 1)
    m1 = jnp.max(p, axis=-1, keepdims=True)
    i1 = jnp.min(jnp.where(p == m1, col, _E), axis=-1, keepdims=True)
    masked = jnp.where(col == i1, -1.0, p)
    m2 = jnp.max(masked, axis=-1, keepdims=True)
    i2 = jnp.min(jnp.where(masked == m2, col, _E), axis=-1, keepdims=True)

    tw_ref[...] = jnp.concatenate([m1, m2], axis=-1)
    ti_ref[...] = jnp.concatenate([i1, i2], axis=-1)


@functools.partial(jax.jit, static_argnames=())
def _router(x, W, b):
    grid = (_TOKENS // _T,)
    return pl.pallas_call(
        _router_body,
        grid=grid,
        in_specs=[
            pl.BlockSpec((_T, _D), lambda i: (i, 0)),
            pl.BlockSpec((_E, _D), lambda i: (0, 0)),
            pl.BlockSpec((1, _E), lambda i: (0, 0)),
        ],
        out_specs=[
            pl.BlockSpec((_T, 2), lambda i: (i, 0)),
            pl.BlockSpec((_T, 2), lambda i: (i, 0)),
        ],
        out_shape=[
            jax.ShapeDtypeStruct((_TOKENS, 2), jnp.float32),
            jax.ShapeDtypeStruct((_TOKENS, 2), jnp.int32),
        ],
    )(x, W, b.reshape(1, _E))


def kernel(x, k, W, b):
    tw, ti = _router(x, W, b)
    ti = ti + (jnp.asarray(k, dtype=ti.dtype) - 2)
    return (tw, ti)


# speedup vs baseline: 1.2527x; 1.0282x over previous
import functools

import jax
import jax.numpy as jnp
from jax.experimental import pallas as pl
from jax.experimental.pallas import tpu as pltpu

_TOKENS = 16384
_D = 2048
_E = 64
_T = 1024
_NBUF = 4
_NCHUNK = _TOKENS // _T


def _router_body(x_hbm, w_ref, b_ref, tw_ref, ti_ref, xbuf, sems):
    i = pl.program_id(0)

    def chunk_copy(j, slot):
        return pltpu.make_async_copy(
            x_hbm.at[pl.ds(j * _T, _T), :], xbuf.at[slot], sems.at[slot])

    @pl.when(i == 0)
    def _prime():
        for s in range(_NBUF - 1):
            chunk_copy(s, s).start()

    pref = i + _NBUF - 1

    @pl.when(pref < _NCHUNK)
    def _prefetch():
        chunk_copy(pref, jax.lax.rem(pref, _NBUF)).start()

    slot = jax.lax.rem(i, _NBUF)
    chunk_copy(i, slot).wait()

    logits = jax.lax.dot_general(
        xbuf[slot], w_ref[...], (((1,), (1,)), ((), ())),
        preferred_element_type=jnp.float32)
    logits = logits + b_ref[...]
    m = jnp.max(logits, axis=-1, keepdims=True)
    e = jnp.exp(logits - m)
    p = e / jnp.sum(e, axis=-1, keepdims=True)

    col = jax.lax.broadcasted_iota(jnp.int32, p.shape, 1)
    m1 = jnp.max(p, axis=-1, keepdims=True)
    i1 = jnp.min(jnp.where(p == m1, col, _E), axis=-1, keepdims=True)
    masked = jnp.where(col == i1, -1.0, p)
    m2 = jnp.max(masked, axis=-1, keepdims=True)
    i2 = jnp.min(jnp.where(masked == m2, col, _E), axis=-1, keepdims=True)

    tw_ref[...] = jnp.concatenate([m1, m2], axis=-1)
    ti_ref[...] = jnp.concatenate([i1, i2], axis=-1)


@functools.partial(jax.jit, static_argnames=())
def _router(x, W, b):
    return pl.pallas_call(
        _router_body,
        grid=(_NCHUNK,),
        in_specs=[
            pl.BlockSpec(memory_space=pl.ANY),
            pl.BlockSpec((_E, _D), lambda i: (0, 0)),
            pl.BlockSpec((1, _E), lambda i: (0, 0)),
        ],
        out_specs=[
            pl.BlockSpec((_T, 2), lambda i: (i, 0)),
            pl.BlockSpec((_T, 2), lambda i: (i, 0)),
        ],
        out_shape=[
            jax.ShapeDtypeStruct((_TOKENS, 2), jnp.float32),
            jax.ShapeDtypeStruct((_TOKENS, 2), jnp.int32),
        ],
        scratch_shapes=[
            pltpu.VMEM((_NBUF, _T, _D), jnp.float32),
            pltpu.SemaphoreType.DMA((_NBUF,)),
        ],
    )(x, W, b.reshape(1, _E))


def kernel(x, k, W, b):
    tw, ti = _router(x, W, b)
    ti = ti + (jnp.asarray(k, dtype=ti.dtype) - 2)
    return (tw, ti)
